# baseline (speedup 1.0000x reference)
import functools

import jax
import jax.numpy as jnp
from jax import lax
from jax.experimental import pallas as pl
from jax.experimental.pallas import tpu as pltpu
from jax.experimental.pallas import tpu_sc as plsc

NC, NS, LANES = 2, 16, 16
NW = NC * NS
H_DIM = 512


@functools.partial(jax.jit, static_argnums=(0, 1, 2, 3, 4))
def _pool_sc(E, K, N, D_IMG, D_TXT, idx_flat, feats_img, feats_txt):
    EW = E // NW
    CE = 8
    NCH = EW // CE
    ROWS = CE * K
    NBUF = 2
    inv_k = 1.0 / K

    mesh = plsc.VectorSubcoreMesh(core_axis_name="c", subcore_axis_name="s",
                                  num_cores=NC, num_subcores=NS)

    def _tree_sum(vals):
        while len(vals) > 1:
            nxt = [vals[i] + vals[i + 1] for i in range(0, len(vals) - 1, 2)]
            if len(vals) % 2:
                nxt.append(vals[-1])
            vals = nxt
        return vals[0]

    def body(idx_hbm, img_hbm, txt_hbm, out_img, out_txt,
             idx_all, img_v, txt_v, pimg_v, ptxt_v,
             sem_i0, sem_i1, sem_t0, sem_t1,
             sem_oi0, sem_oi1, sem_ot0, sem_ot1):
        sems_i = (sem_i0, sem_i1)
        sems_t = (sem_t0, sem_t1)
        sems_oi = (sem_oi0, sem_oi1)
        sems_ot = (sem_ot0, sem_ot1)
        wid = lax.axis_index("s") * NC + lax.axis_index("c")
        ebase = wid * EW
        pltpu.sync_copy(idx_hbm.at[pl.ds(ebase * K, EW * K)], idx_all)

        def fetch_img(cc, b):
            idx = idx_all.at[pl.ds(cc * ROWS, ROWS)]
            pltpu.async_copy(img_hbm.at[idx], img_v.at[b], sems_i[b])

        def fetch_txt(cc, b):
            idx = idx_all.at[pl.ds(cc * ROWS, ROWS)]
            pltpu.async_copy(txt_hbm.at[idx], txt_v.at[b], sems_t[b])

        for b in range(NBUF):
            fetch_img(b, b)
            fetch_txt(b, b)

        @pl.loop(0, NCH, step=NBUF)
        def _chunk(c):
            for b in range(NBUF):
                cc = c + b
                e0 = ebase + cc * CE
                idx_b = idx_all.at[pl.ds(cc * ROWS, ROWS)]
                pltpu.make_async_copy(img_hbm.at[idx_b],
                                      img_v.at[b], sems_i[b]).wait()
                @pl.when(cc >= NBUF)
                def _():
                    pltpu.make_async_copy(
                        pimg_v.at[b], out_img.at[pl.ds(e0, CE)],
                        sems_oi[b]).wait()

                @plsc.parallel_loop(0, CE, unroll=4)
                def _eimg(e):
                    r0 = e * K
                    for db in range(D_IMG // LANES):
                        sl = pl.ds(db * LANES, LANES)
                        acc = _tree_sum([img_v[b, r0 + k, sl]
                                         for k in range(K)])
                        pimg_v[b, e, sl] = acc * inv_k

                @pl.when(cc + NBUF < NCH)
                def _():
                    fetch_img(cc + NBUF, b)

                pltpu.async_copy(pimg_v.at[b], out_img.at[pl.ds(e0, CE)],
                                 sems_oi[b])
                pltpu.make_async_copy(txt_hbm.at[idx_b],
                                      txt_v.at[b], sems_t[b]).wait()

                @pl.when(cc >= NBUF)
                def _():
                    pltpu.make_async_copy(
                        ptxt_v.at[b], out_txt.at[pl.ds(e0, CE)],
                        sems_ot[b]).wait()

                @plsc.parallel_loop(0, CE, unroll=4)
                def _etxt(e):
                    r0 = e * K
                    for db in range(D_TXT // LANES):
                        sl = pl.ds(db * LANES, LANES)
                        acc = _tree_sum([txt_v[b, r0 + k, sl]
                                         for k in range(K)])
                        ptxt_v[b, e, sl] = acc * inv_k

                @pl.when(cc + NBUF < NCH)
                def _():
                    fetch_txt(cc + NBUF, b)

                pltpu.async_copy(ptxt_v.at[b], out_txt.at[pl.ds(e0, CE)],
                                 sems_ot[b])

        for b in range(NBUF):
            e_last = ebase + (NCH - NBUF + b) * CE
            pltpu.make_async_copy(pimg_v.at[b],
                                  out_img.at[pl.ds(e_last, CE)],
                                  sems_oi[b]).wait()
            pltpu.make_async_copy(ptxt_v.at[b],
                                  out_txt.at[pl.ds(e_last, CE)],
                                  sems_ot[b]).wait()

    fn = pl.kernel(
        body,
        out_type=(jax.ShapeDtypeStruct((E, D_IMG), jnp.float32),
                  jax.ShapeDtypeStruct((E, D_TXT), jnp.float32)),
        mesh=mesh,
        scratch_types=[
            pltpu.VMEM((EW * K,), jnp.int32),
            pltpu.VMEM((NBUF, ROWS, D_IMG), jnp.float32),
            pltpu.VMEM((NBUF, ROWS, D_TXT), jnp.float32),
            pltpu.VMEM((NBUF, CE, D_IMG), jnp.float32),
            pltpu.VMEM((NBUF, CE, D_TXT), jnp.float32),
        ] + [pltpu.SemaphoreType.DMA] * 8,
    )
    return fn(idx_flat, feats_img, feats_txt)


def _mlp_body(pimg_ref, ptxt_ref, W1i_ref, b1i_ref, W1t_ref, b1t_ref,
              w2cat_ref, b2row_ref, sel_ref, scores_ref, mask_ref):
    dn = (((1,), (0,)), ((), ()))
    hi = jnp.maximum(
        lax.dot_general(pimg_ref[...].astype(jnp.bfloat16), W1i_ref[...], dn,
                        preferred_element_type=jnp.float32) + b1i_ref[...], 0.0)
    ht = jnp.maximum(
        lax.dot_general(ptxt_ref[...].astype(jnp.bfloat16), W1t_ref[...], dn,
                        preferred_element_type=jnp.float32) + b1t_ref[...], 0.0)
    s2 = (lax.dot_general(hi.astype(jnp.bfloat16), w2cat_ref[:H_DIM], dn,
                          preferred_element_type=jnp.float32)
          + lax.dot_general(ht.astype(jnp.bfloat16), w2cat_ref[H_DIM:], dn,
                            preferred_element_type=jnp.float32))
    e = jnp.sum((s2 + b2row_ref[...]) * sel_ref[...], axis=1)
    sc = jax.nn.sigmoid(e)
    scores_ref[...] = sc
    mask_ref[...] = sc > 0.5


@functools.partial(jax.jit, static_argnums=(0, 1, 2, 3))
def _mlp_tc(E, D_IMG, D_TXT, H, pooled_img, pooled_txt,
            W1i, b1i, W1t, b1t, w2cat, b2row, sel):
    BLK = 2048
    NB = E // BLK
    return pl.pallas_call(
        _mlp_body,
        grid=(NB,),
        in_specs=[
            pl.BlockSpec((BLK, D_IMG), lambda i: (i, 0)),
            pl.BlockSpec((BLK, D_TXT), lambda i: (i, 0)),
            pl.BlockSpec((D_IMG, H), lambda i: (0, 0)),
            pl.BlockSpec((1, H), lambda i: (0, 0)),
            pl.BlockSpec((D_TXT, H), lambda i: (0, 0)),
            pl.BlockSpec((1, H), lambda i: (0, 0)),
            pl.BlockSpec((2 * H, 128), lambda i: (0, 0)),
            pl.BlockSpec((1, 128), lambda i: (0, 0)),
            pl.BlockSpec((1, 128), lambda i: (0, 0)),
        ],
        out_specs=[
            pl.BlockSpec((BLK,), lambda i: (i,)),
            pl.BlockSpec((BLK,), lambda i: (i,)),
        ],
        out_shape=[
            jax.ShapeDtypeStruct((E,), jnp.float32),
            jax.ShapeDtypeStruct((E,), jnp.bool_),
        ],
        compiler_params=pltpu.CompilerParams(
            dimension_semantics=("arbitrary",)),
    )(pooled_img, pooled_txt, W1i, b1i, W1t, b1t, w2cat, b2row, sel)


def kernel(hyperedges, features_image, features_text, W1_image, b1_image,
           W2_image, b2_image, W1_text, b1_text, W2_text, b2_text, alpha):
    E, K = hyperedges.shape
    N, D_IMG = features_image.shape
    _, D_TXT = features_text.shape
    H = W1_image.shape[1]

    idx_flat = hyperedges.reshape(E * K).astype(jnp.int32)

    w = jax.nn.softmax(alpha, axis=0)
    bf16 = jnp.bfloat16
    w2cat = (jnp.zeros((2 * H, 128), jnp.float32)
             .at[:H, 0].set(W2_image[:, 0])
             .at[H:, 1].set(W2_text[:, 0])).astype(bf16)
    b2row = (jnp.zeros((1, 128), jnp.float32)
             .at[0, 0].set(b2_image[0])
             .at[0, 1].set(b2_text[0]))
    sel = (jnp.zeros((1, 128), jnp.float32)
           .at[0, 0].set(w[0])
           .at[0, 1].set(w[1]))
    CHUNKS = 2
    EC = E // CHUNKS
    scores_parts, mask_parts = [], []
    for c in range(CHUNKS):
        idx_c = lax.slice_in_dim(idx_flat, c * EC * K, (c + 1) * EC * K)
        pi_c, pt_c = _pool_sc(EC, K, N, D_IMG, D_TXT,
                              idx_c, features_image, features_text)
        s_c, m_c = _mlp_tc(EC, D_IMG, D_TXT, H, pi_c, pt_c,
                           W1_image.astype(bf16), b1_image.reshape(1, H),
                           W1_text.astype(bf16), b1_text.reshape(1, H),
                           w2cat, b2row, sel)
        scores_parts.append(s_c)
        mask_parts.append(m_c)
    scores = jnp.concatenate(scores_parts)
    mask = jnp.concatenate(mask_parts)
    return (mask, scores)


# speedup vs baseline: 1.8522x; 1.8522x over previous
import functools

import jax
import jax.numpy as jnp
from jax import lax
from jax.experimental import pallas as pl
from jax.experimental.pallas import tpu as pltpu
from jax.experimental.pallas import tpu_sc as plsc

NC, NS, LANES = 2, 16, 16
NW = NC * NS
H_DIM = 512


@functools.partial(jax.jit, static_argnums=(0, 1, 2, 3, 4))
def _pool_sc(E, K, N, D_IMG, D_TXT, idx_flat, feats_img, feats_txt):
    EW = E // NW
    CE = 8
    NCH = EW // CE
    ROWS = CE * K
    NBUF = 2
    inv_k = 1.0 / K

    mesh = plsc.VectorSubcoreMesh(core_axis_name="c", subcore_axis_name="s",
                                  num_cores=NC, num_subcores=NS)

    def _tree_sum(vals):
        while len(vals) > 1:
            nxt = [vals[i] + vals[i + 1] for i in range(0, len(vals) - 1, 2)]
            if len(vals) % 2:
                nxt.append(vals[-1])
            vals = nxt
        return vals[0]

    def body(idx_hbm, img_hbm, txt_hbm, out_img, out_txt,
             idx_all, img_v, txt_v, pimg_v, ptxt_v,
             sem_i0, sem_i1, sem_t0, sem_t1,
             sem_oi0, sem_oi1, sem_ot0, sem_ot1):
        sems_i = (sem_i0, sem_i1)
        sems_t = (sem_t0, sem_t1)
        sems_oi = (sem_oi0, sem_oi1)
        sems_ot = (sem_ot0, sem_ot1)
        wid = lax.axis_index("s") * NC + lax.axis_index("c")
        ebase = wid * EW
        pltpu.sync_copy(idx_hbm.at[pl.ds(ebase * K, EW * K)], idx_all)

        def fetch_img(cc, b):
            idx = idx_all.at[pl.ds(cc * ROWS, ROWS)]
            pltpu.async_copy(img_hbm.at[idx], img_v.at[b], sems_i[b])

        def fetch_txt(cc, b):
            idx = idx_all.at[pl.ds(cc * ROWS, ROWS)]
            pltpu.async_copy(txt_hbm.at[idx], txt_v.at[b], sems_t[b])

        for b in range(NBUF):
            fetch_img(b, b)
            fetch_txt(b, b)

        @pl.loop(0, NCH, step=NBUF)
        def _chunk(c):
            for b in range(NBUF):
                cc = c + b
                e0 = ebase + cc * CE
                idx_b = idx_all.at[pl.ds(cc * ROWS, ROWS)]
                pltpu.make_async_copy(img_hbm.at[idx_b],
                                      img_v.at[b], sems_i[b]).wait()
                @pl.when(cc >= NBUF)
                def _():
                    pltpu.make_async_copy(
                        pimg_v.at[b], out_img.at[pl.ds(e0, CE)],
                        sems_oi[b]).wait()

                @plsc.parallel_loop(0, CE, unroll=2)
                def _eimg(e):
                    r0 = e * K
                    for db in range(D_IMG // LANES):
                        sl = pl.ds(db * LANES, LANES)
                        acc = _tree_sum([img_v[b, r0 + k, sl]
                                         for k in range(K)])
                        pimg_v[b, e, sl] = acc * inv_k

                @pl.when(cc + NBUF < NCH)
                def _():
                    fetch_img(cc + NBUF, b)

                pltpu.async_copy(pimg_v.at[b], out_img.at[pl.ds(e0, CE)],
                                 sems_oi[b])
                pltpu.make_async_copy(txt_hbm.at[idx_b],
                                      txt_v.at[b], sems_t[b]).wait()

                @pl.when(cc >= NBUF)
                def _():
                    pltpu.make_async_copy(
                        ptxt_v.at[b], out_txt.at[pl.ds(e0, CE)],
                        sems_ot[b]).wait()

                @plsc.parallel_loop(0, CE, unroll=2)
                def _etxt(e):
                    r0 = e * K
                    for db in range(D_TXT // LANES):
                        sl = pl.ds(db * LANES, LANES)
                        acc = _tree_sum([txt_v[b, r0 + k, sl]
                                         for k in range(K)])
                        ptxt_v[b, e, sl] = acc * inv_k

                @pl.when(cc + NBUF < NCH)
                def _():
                    fetch_txt(cc + NBUF, b)

                pltpu.async_copy(ptxt_v.at[b], out_txt.at[pl.ds(e0, CE)],
                                 sems_ot[b])

        for b in range(NBUF):
            e_last = ebase + (NCH - NBUF + b) * CE
            pltpu.make_async_copy(pimg_v.at[b],
                                  out_img.at[pl.ds(e_last, CE)],
                                  sems_oi[b]).wait()
            pltpu.make_async_copy(ptxt_v.at[b],
                                  out_txt.at[pl.ds(e_last, CE)],
                                  sems_ot[b]).wait()

    fn = pl.kernel(
        body,
        out_type=(jax.ShapeDtypeStruct((E, D_IMG), jnp.float32),
                  jax.ShapeDtypeStruct((E, D_TXT), jnp.float32)),
        mesh=mesh,
        scratch_types=[
            pltpu.VMEM((EW * K,), jnp.int32),
            pltpu.VMEM((NBUF, ROWS, D_IMG), jnp.float32),
            pltpu.VMEM((NBUF, ROWS, D_TXT), jnp.float32),
            pltpu.VMEM((NBUF, CE, D_IMG), jnp.float32),
            pltpu.VMEM((NBUF, CE, D_TXT), jnp.float32),
        ] + [pltpu.SemaphoreType.DMA] * 8,
    )
    return fn(idx_flat, feats_img, feats_txt)


def _mlp_body(pimg_ref, ptxt_ref, W1i_ref, b1i_ref, W1t_ref, b1t_ref,
              w2cat_ref, b2row_ref, sel_ref, scores_ref, mask_ref):
    dn = (((1,), (0,)), ((), ()))
    hi = jnp.maximum(
        lax.dot_general(pimg_ref[...].astype(jnp.bfloat16), W1i_ref[...], dn,
                        preferred_element_type=jnp.float32) + b1i_ref[...], 0.0)
    ht = jnp.maximum(
        lax.dot_general(ptxt_ref[...].astype(jnp.bfloat16), W1t_ref[...], dn,
                        preferred_element_type=jnp.float32) + b1t_ref[...], 0.0)
    s2 = (lax.dot_general(hi.astype(jnp.bfloat16), w2cat_ref[:H_DIM], dn,
                          preferred_element_type=jnp.float32)
          + lax.dot_general(ht.astype(jnp.bfloat16), w2cat_ref[H_DIM:], dn,
                            preferred_element_type=jnp.float32))
    e = jnp.sum((s2 + b2row_ref[...]) * sel_ref[...], axis=1)
    sc = jax.nn.sigmoid(e)
    scores_ref[...] = sc
    mask_ref[...] = sc > 0.5


@functools.partial(jax.jit, static_argnums=(0, 1, 2, 3))
def _mlp_tc(E, D_IMG, D_TXT, H, pooled_img, pooled_txt,
            W1i, b1i, W1t, b1t, w2cat, b2row, sel):
    BLK = 2048
    NB = E // BLK
    return pl.pallas_call(
        _mlp_body,
        grid=(NB,),
        in_specs=[
            pl.BlockSpec((BLK, D_IMG), lambda i: (i, 0)),
            pl.BlockSpec((BLK, D_TXT), lambda i: (i, 0)),
            pl.BlockSpec((D_IMG, H), lambda i: (0, 0)),
            pl.BlockSpec((1, H), lambda i: (0, 0)),
            pl.BlockSpec((D_TXT, H), lambda i: (0, 0)),
            pl.BlockSpec((1, H), lambda i: (0, 0)),
            pl.BlockSpec((2 * H, 128), lambda i: (0, 0)),
            pl.BlockSpec((1, 128), lambda i: (0, 0)),
            pl.BlockSpec((1, 128), lambda i: (0, 0)),
        ],
        out_specs=[
            pl.BlockSpec((BLK,), lambda i: (i,)),
            pl.BlockSpec((BLK,), lambda i: (i,)),
        ],
        out_shape=[
            jax.ShapeDtypeStruct((E,), jnp.float32),
            jax.ShapeDtypeStruct((E,), jnp.bool_),
        ],
        compiler_params=pltpu.CompilerParams(
            dimension_semantics=("arbitrary",)),
    )(pooled_img, pooled_txt, W1i, b1i, W1t, b1t, w2cat, b2row, sel)


def kernel(hyperedges, features_image, features_text, W1_image, b1_image,
           W2_image, b2_image, W1_text, b1_text, W2_text, b2_text, alpha):
    E, K = hyperedges.shape
    N, D_IMG = features_image.shape
    _, D_TXT = features_text.shape
    H = W1_image.shape[1]

    idx_flat = hyperedges.reshape(E * K).astype(jnp.int32)

    w = jax.nn.softmax(alpha, axis=0)
    bf16 = jnp.bfloat16
    w2cat = (jnp.zeros((2 * H, 128), jnp.float32)
             .at[:H, 0].set(W2_image[:, 0])
             .at[H:, 1].set(W2_text[:, 0])).astype(bf16)
    b2row = (jnp.zeros((1, 128), jnp.float32)
             .at[0, 0].set(b2_image[0])
             .at[0, 1].set(b2_text[0]))
    sel = (jnp.zeros((1, 128), jnp.float32)
           .at[0, 0].set(w[0])
           .at[0, 1].set(w[1]))
    CHUNKS = 2
    EC = E // CHUNKS
    scores_parts, mask_parts = [], []
    for c in range(CHUNKS):
        idx_c = lax.slice_in_dim(idx_flat, c * EC * K, (c + 1) * EC * K)
        pi_c, pt_c = _pool_sc(EC, K, N, D_IMG, D_TXT,
                              idx_c, features_image, features_text)
        s_c, m_c = _mlp_tc(EC, D_IMG, D_TXT, H, pi_c, pt_c,
                           W1_image.astype(bf16), b1_image.reshape(1, H),
                           W1_text.astype(bf16), b1_text.reshape(1, H),
                           w2cat, b2row, sel)
        scores_parts.append(s_c)
        mask_parts.append(m_c)
    scores = jnp.concatenate(scores_parts)
    mask = jnp.concatenate(mask_parts)
    return (mask, scores)
